# baseline (speedup 1.0000x reference)
import functools

import jax
import jax.numpy as jnp
from jax import lax
from jax.experimental import pallas as pl
from jax.experimental.pallas import tpu as pltpu
from jax.experimental.pallas import tpu_sc as plsc

SEQ_SIZE = 200
EMBED_DIM = 64
BATCH = 4096


def _make_sc_gather(V, D, B):
    info = plsc.get_sparse_core_info()
    NC, NS = info.num_cores, info.num_subcores
    NW = NC * NS
    assert B % (8 * NW) == 0
    b_per_w = B // NW
    mesh = plsc.VectorSubcoreMesh(core_axis_name="c", subcore_axis_name="s")

    @functools.partial(
        pl.kernel,
        mesh=mesh,
        out_type=jax.ShapeDtypeStruct((B, D), jnp.float32),
        scratch_types=[
            pltpu.VMEM((b_per_w,), jnp.int32),
            pltpu.VMEM((b_per_w, D), jnp.float32),
            pltpu.SemaphoreType.DMA,
        ],
        compiler_params=pltpu.CompilerParams(use_tc_tiling_on_sc=False),
    )
    def gather_kernel(table_hbm, idx_hbm, out_hbm, idx_v, rows_v, sem):
        wid = lax.axis_index("s") * NC + lax.axis_index("c")
        base = wid * b_per_w
        pltpu.sync_copy(idx_hbm.at[pl.ds(base, b_per_w)], idx_v)
        pltpu.async_copy(table_hbm.at[idx_v], rows_v, sem).wait()
        pltpu.sync_copy(rows_v, out_hbm.at[pl.ds(base, b_per_w)])

    return gather_kernel


def _bcast_add_body(gath_ref, pos_ref, out_ref):
    g = gath_ref[...]
    p = pos_ref[...]
    out_ref[...] = g[:, None, :] + p[None, :, :]


def kernel(inputs, token_table, pos_table):
    V, D = token_table.shape
    B = inputs.shape[0]
    idx = inputs.reshape(B).astype(jnp.int32)

    gathered = _make_sc_gather(V, D, B)(token_table, idx)

    BB = 256
    out = pl.pallas_call(
        _bcast_add_body,
        grid=(B // BB,),
        in_specs=[
            pl.BlockSpec((BB, D), lambda i: (i, 0)),
            pl.BlockSpec((SEQ_SIZE, D), lambda i: (0, 0)),
        ],
        out_specs=pl.BlockSpec((BB, SEQ_SIZE, D), lambda i: (i, 0, 0)),
        out_shape=jax.ShapeDtypeStruct((B, SEQ_SIZE, D), jnp.float32),
    )(gathered, pos_table)
    return out


# speedup vs baseline: 1.5864x; 1.5864x over previous
import functools

import jax
import jax.numpy as jnp
from jax import lax
from jax.experimental import pallas as pl
from jax.experimental.pallas import tpu as pltpu
from jax.experimental.pallas import tpu_sc as plsc

SEQ_SIZE = 200
EMBED_DIM = 64
BATCH = 4096


def _make_sc_gather(V, D, B):
    info = plsc.get_sparse_core_info()
    NC, NS = info.num_cores, info.num_subcores
    NW = NC * NS
    assert B % (8 * NW) == 0
    b_per_w = B // NW
    mesh = plsc.VectorSubcoreMesh(core_axis_name="c", subcore_axis_name="s")

    @functools.partial(
        pl.kernel,
        mesh=mesh,
        out_type=jax.ShapeDtypeStruct((B, D), jnp.float32),
        scratch_types=[
            pltpu.VMEM((b_per_w,), jnp.int32),
            pltpu.VMEM((b_per_w, D), jnp.float32),
            pltpu.SemaphoreType.DMA,
        ],
        compiler_params=pltpu.CompilerParams(use_tc_tiling_on_sc=False),
    )
    def gather_kernel(table_hbm, idx_hbm, out_hbm, idx_v, rows_v, sem):
        wid = lax.axis_index("s") * NC + lax.axis_index("c")
        base = wid * b_per_w
        pltpu.sync_copy(idx_hbm.at[pl.ds(base, b_per_w)], idx_v)
        pltpu.async_copy(table_hbm.at[idx_v], rows_v, sem).wait()
        pltpu.sync_copy(rows_v, out_hbm.at[pl.ds(base, b_per_w)])

    return gather_kernel


def _bcast_add_body(gath_ref, pos_ref, out_ref):
    g = gath_ref[...]
    p = pos_ref[...]
    out_ref[...] = g[:, None, :] + p[None, :, :]


def kernel(inputs, token_table, pos_table):
    V, D = token_table.shape
    B = inputs.shape[0]
    idx = inputs.reshape(B).astype(jnp.int32)

    gathered = jnp.take(token_table, idx, axis=0)

    BB = 256
    out = pl.pallas_call(
        _bcast_add_body,
        grid=(B // BB,),
        in_specs=[
            pl.BlockSpec((BB, D), lambda i: (i, 0)),
            pl.BlockSpec((SEQ_SIZE, D), lambda i: (0, 0)),
        ],
        out_specs=pl.BlockSpec((BB, SEQ_SIZE, D), lambda i: (i, 0, 0)),
        out_shape=jax.ShapeDtypeStruct((B, SEQ_SIZE, D), jnp.float32),
    )(gathered, pos_table)
    return out


# speedup vs baseline: 1.5884x; 1.0013x over previous
import functools

import jax
import jax.numpy as jnp
from jax import lax
from jax.experimental import pallas as pl
from jax.experimental.pallas import tpu as pltpu
from jax.experimental.pallas import tpu_sc as plsc

SEQ_SIZE = 200
EMBED_DIM = 64
BATCH = 4096


def _make_sc_gather(V, D, B):
    info = plsc.get_sparse_core_info()
    NC, NS = info.num_cores, info.num_subcores
    NW = NC * NS
    assert B % (8 * NW) == 0
    b_per_w = B // NW
    mesh = plsc.VectorSubcoreMesh(core_axis_name="c", subcore_axis_name="s")

    @functools.partial(
        pl.kernel,
        mesh=mesh,
        out_type=jax.ShapeDtypeStruct((B, D), jnp.float32),
        scratch_types=[
            pltpu.VMEM((b_per_w,), jnp.int32),
            pltpu.VMEM((b_per_w, D), jnp.float32),
            pltpu.SemaphoreType.DMA,
        ],
        compiler_params=pltpu.CompilerParams(use_tc_tiling_on_sc=False),
    )
    def gather_kernel(table_hbm, idx_hbm, out_hbm, idx_v, rows_v, sem):
        wid = lax.axis_index("s") * NC + lax.axis_index("c")
        base = wid * b_per_w
        pltpu.sync_copy(idx_hbm.at[pl.ds(base, b_per_w)], idx_v)
        pltpu.async_copy(table_hbm.at[idx_v], rows_v, sem).wait()
        pltpu.sync_copy(rows_v, out_hbm.at[pl.ds(base, b_per_w)])

    return gather_kernel


_BB = 64
_NBUF = 4


def _bcast_add_body(gath_ref, pos_ref, out_ref, buf_ref, sem_ref):
    i = pl.program_id(0)
    n = pl.num_programs(0)
    g = gath_ref[...]
    p = pos_ref[...]
    val = g[:, None, :] + p[None, :, :]
    slot = jax.lax.rem(i, _NBUF)
    for k in range(_NBUF):
        @pl.when(slot == k)
        def _(k=k):
            @pl.when(i >= _NBUF)
            def _():
                pltpu.make_async_copy(
                    buf_ref.at[k], out_ref.at[pl.ds(i * _BB, _BB)],
                    sem_ref.at[k]).wait()
            buf_ref[k] = val
            pltpu.make_async_copy(
                buf_ref.at[k], out_ref.at[pl.ds(i * _BB, _BB)],
                sem_ref.at[k]).start()

    @pl.when(i == n - 1)
    def _():
        for k in range(_NBUF):
            pltpu.make_async_copy(
                buf_ref.at[k], out_ref.at[pl.ds(0, _BB)],
                sem_ref.at[k]).wait()


def kernel(inputs, token_table, pos_table):
    V, D = token_table.shape
    B = inputs.shape[0]
    idx = inputs.reshape(B).astype(jnp.int32)

    gathered = jnp.take(token_table, idx, axis=0)

    out = pl.pallas_call(
        _bcast_add_body,
        grid=(B // _BB,),
        in_specs=[
            pl.BlockSpec((_BB, D), lambda i: (i, 0)),
            pl.BlockSpec((SEQ_SIZE, D), lambda i: (0, 0)),
        ],
        out_specs=pl.BlockSpec(memory_space=pl.ANY),
        out_shape=jax.ShapeDtypeStruct((B, SEQ_SIZE, D), jnp.float32),
        scratch_shapes=[
            pltpu.VMEM((_NBUF, _BB, SEQ_SIZE, D), jnp.float32),
            pltpu.SemaphoreType.DMA((_NBUF,)),
        ],
    )(gathered, pos_table)
    return out


# speedup vs baseline: 2.0531x; 1.2926x over previous
import functools

import jax
import jax.numpy as jnp
from jax import lax
from jax.experimental import pallas as pl
from jax.experimental.pallas import tpu as pltpu
from jax.experimental.pallas import tpu_sc as plsc

SEQ_SIZE = 200
EMBED_DIM = 64
BATCH = 4096


def _make_sc_gather(V, D, B):
    info = plsc.get_sparse_core_info()
    NC, NS = info.num_cores, info.num_subcores
    NW = NC * NS
    assert B % (8 * NW) == 0
    b_per_w = B // NW
    mesh = plsc.VectorSubcoreMesh(core_axis_name="c", subcore_axis_name="s")

    @functools.partial(
        pl.kernel,
        mesh=mesh,
        out_type=jax.ShapeDtypeStruct((B, D), jnp.float32),
        scratch_types=[
            pltpu.VMEM((b_per_w,), jnp.int32),
            pltpu.VMEM((b_per_w, D), jnp.float32),
            pltpu.SemaphoreType.DMA,
        ],
        compiler_params=pltpu.CompilerParams(use_tc_tiling_on_sc=False),
    )
    def gather_kernel(table_hbm, idx_hbm, out_hbm, idx_v, rows_v, sem):
        wid = lax.axis_index("s") * NC + lax.axis_index("c")
        base = wid * b_per_w
        pltpu.sync_copy(idx_hbm.at[pl.ds(base, b_per_w)], idx_v)
        pltpu.async_copy(table_hbm.at[idx_v], rows_v, sem).wait()
        pltpu.sync_copy(rows_v, out_hbm.at[pl.ds(base, b_per_w)])

    return gather_kernel


_BB = 128


def _bcast_add_body(gath_ref, pos_ref, out_ref):
    g = gath_ref[...]
    p = pos_ref[...]
    out_ref[...] = g[:, None, :] + p[None, :, :]


def kernel(inputs, token_table, pos_table):
    V, D = token_table.shape
    B = inputs.shape[0]
    idx = inputs.reshape(B).astype(jnp.int32)

    gathered = jnp.take(token_table, idx, axis=0)

    g2 = jnp.concatenate([gathered, gathered], axis=1)
    pos2 = pos_table.reshape(SEQ_SIZE // 2, 2 * D)
    out2 = pl.pallas_call(
        _bcast_add_body,
        grid=(B // _BB,),
        in_specs=[
            pl.BlockSpec((_BB, 2 * D), lambda i: (i, 0)),
            pl.BlockSpec((SEQ_SIZE // 2, 2 * D), lambda i: (0, 0)),
        ],
        out_specs=pl.BlockSpec((_BB, SEQ_SIZE // 2, 2 * D), lambda i: (i, 0, 0)),
        out_shape=jax.ShapeDtypeStruct((B, SEQ_SIZE // 2, 2 * D), jnp.float32),
    )(g2, pos2)
    return out2.reshape(B, SEQ_SIZE, D)


# speedup vs baseline: 3.2764x; 1.5958x over previous
import functools

import jax
import jax.numpy as jnp
from jax import lax
from jax.experimental import pallas as pl
from jax.experimental.pallas import tpu as pltpu
from jax.experimental.pallas import tpu_sc as plsc

SEQ_SIZE = 200
EMBED_DIM = 64
BATCH = 4096

_BBL = 256


def _bcast_add_body(g_ref, posb_ref, out_ref):
    g = g_ref[...]
    pb = posb_ref[...]
    out_ref[...] = pb + g[None, :, :]


def kernel(inputs, token_table, pos_table):
    V, D = token_table.shape
    B = inputs.shape[0]
    idx = inputs.reshape(B).astype(jnp.int32)

    gathered = jnp.take(token_table, idx, axis=0)
    gT = gathered.T

    posB = jnp.broadcast_to(pos_table[:, :, None], (SEQ_SIZE, D, _BBL))
    P = pl.pallas_call(
        _bcast_add_body,
        grid=(B // _BBL,),
        in_specs=[
            pl.BlockSpec((D, _BBL), lambda i: (0, i)),
            pl.BlockSpec((SEQ_SIZE, D, _BBL), lambda i: (0, 0, 0)),
        ],
        out_specs=pl.BlockSpec((SEQ_SIZE, D, _BBL), lambda i: (0, 0, i)),
        out_shape=jax.ShapeDtypeStruct((SEQ_SIZE, D, B), jnp.float32),
    )(gT, posB)
    return jnp.transpose(P, (2, 0, 1))
